# baseline (speedup 1.0000x reference)
import jax
import jax.numpy as jnp
from jax.experimental import pallas as pl
from jax.experimental.pallas import tpu as pltpu

_NROWS = 26
_ROW = 16384 * 32


def _gather_body(map_ref, in_ref, out_ref, sem):
    copies = []
    for i in range(_NROWS):
        c = pltpu.make_async_copy(in_ref.at[map_ref[i]], out_ref.at[i], sem)
        c.start()
        copies.append(c)
    for c in copies:
        c.wait()


def kernel(mamdani_output, mapping):
    src = mamdani_output.reshape(100, _ROW)
    idx = mapping.reshape(_NROWS).astype(jnp.int32)
    out = pl.pallas_call(
        _gather_body,
        grid_spec=pltpu.PrefetchScalarGridSpec(
            num_scalar_prefetch=1,
            grid=(1,),
            in_specs=[pl.BlockSpec(memory_space=pl.MemorySpace.ANY)],
            out_specs=pl.BlockSpec(memory_space=pl.MemorySpace.ANY),
            scratch_shapes=[pltpu.SemaphoreType.DMA],
        ),
        out_shape=jax.ShapeDtypeStruct((_NROWS, _ROW), jnp.float32),
    )(idx, src)
    return out.reshape(_NROWS, 1, 16384, 32)


# speedup vs baseline: 3.3971x; 3.3971x over previous
import jax
import jax.numpy as jnp
from jax.experimental import pallas as pl
from jax.experimental.pallas import tpu as pltpu

_NROWS = 26
_SUB = 4096
_SPLIT = 4
_BLK = _SUB // _SPLIT


def _copy_body(map_ref, in_ref, out_ref):
    out_ref[...] = in_ref[...]


def kernel(mamdani_output, mapping):
    src = mamdani_output.reshape(100, _SUB, 128)
    idx = mapping.reshape(_NROWS).astype(jnp.int32)
    out = pl.pallas_call(
        _copy_body,
        grid_spec=pltpu.PrefetchScalarGridSpec(
            num_scalar_prefetch=1,
            grid=(_NROWS, _SPLIT),
            in_specs=[pl.BlockSpec((1, _BLK, 128), lambda i, j, m: (m[i], j, 0))],
            out_specs=pl.BlockSpec((1, _BLK, 128), lambda i, j, m: (i, j, 0)),
        ),
        out_shape=jax.ShapeDtypeStruct((_NROWS, _SUB, 128), jnp.float32),
    )(idx, src)
    return out.reshape(_NROWS, 1, 16384, 32)
